# baseline (speedup 1.0000x reference)
import jax
import jax.numpy as jnp
from jax.experimental import pallas as pl
from jax.experimental.pallas import tpu as pltpu

DIM = 1024
NUM_EXPERTS = 8
NUM_HEADS = 16
TOP_K = 2
DH = DIM // NUM_HEADS
SCALE = DH ** (-0.5)


def _top2_weights(logits):
    T, E = logits.shape
    m = jnp.max(logits, axis=-1, keepdims=True)
    p = jnp.exp(logits - m)
    probs = p / jnp.sum(p, axis=-1, keepdims=True)

    idx = jax.lax.broadcasted_iota(jnp.int32, (T, E), 1)
    big = jnp.int32(E)
    i1 = jnp.min(jnp.where(logits == m, idx, big), axis=-1, keepdims=True)
    mask1 = idx == i1
    logits2 = jnp.where(mask1, -jnp.inf, logits)
    max2 = jnp.max(logits2, axis=-1, keepdims=True)
    i2 = jnp.min(jnp.where(logits2 == max2, idx, big), axis=-1, keepdims=True)
    mask2 = idx == i2
    return jnp.where(mask1 | mask2, probs, 0.0)


def _qkv_body(x_ref, wg_ref, bg_ref, wqkv_ref, qkv_ref, w8_ref):
    e = pl.program_id(0)
    t = pl.program_id(1)
    TB = x_ref.shape[0]
    rows = pl.ds(t * TB, TB)
    x = x_ref[...]

    @pl.when(e == 0)
    def _gate():
        logits = jnp.dot(x, wg_ref[...],
                         preferred_element_type=jnp.float32) + bg_ref[...]
        w8_ref[rows, :] = _top2_weights(logits)

    w8 = w8_ref[rows, :]
    lane = jax.lax.broadcasted_iota(jnp.int32, w8.shape, 1)
    w_e = jnp.sum(jnp.where(lane == e, w8, 0.0), axis=1, keepdims=True)
    contrib = w_e * jnp.dot(x.astype(jnp.bfloat16),
                            wqkv_ref[0].astype(jnp.bfloat16),
                            preferred_element_type=jnp.float32)

    @pl.when(e == 0)
    def _init():
        qkv_ref[rows, :] = contrib

    @pl.when(e > 0)
    def _acc():
        qkv_ref[rows, :] = qkv_ref[rows, :] + contrib


def _attn_body(qkv_ref, wp_ref, bp_ref, out_ref, att_ref):
    qkv = qkv_ref[...]
    T = qkv.shape[0]
    q = qkv[:, :DIM]
    k = qkv[:, DIM:2 * DIM]
    v = qkv[:, 2 * DIM:]

    r = jax.lax.broadcasted_iota(jnp.int32, (DIM, NUM_HEADS), 0)
    c = jax.lax.broadcasted_iota(jnp.int32, (DIM, NUM_HEADS), 1)
    S = (r // DH == c).astype(jnp.float32)

    for i in range(NUM_HEADS):
        qi = q[:, i * DH:(i + 1) * DH]
        qrep = jnp.concatenate([qi] * NUM_HEADS, axis=1)
        logits = jnp.dot(qrep * k, S,
                         preferred_element_type=jnp.float32) * SCALE
        logits = logits - jnp.max(logits, axis=-1, keepdims=True)
        w = jnp.exp(logits)
        w = w / jnp.sum(w, axis=-1, keepdims=True)
        out_i = jnp.zeros((T, DH), dtype=jnp.float32)
        for j in range(NUM_HEADS):
            out_i = out_i + w[:, j:j + 1] * v[:, j * DH:(j + 1) * DH]
        att_ref[:, i * DH:(i + 1) * DH] = out_i

    out_ref[...] = jnp.dot(att_ref[...].astype(jnp.bfloat16),
                           wp_ref[...].astype(jnp.bfloat16),
                           preferred_element_type=jnp.float32) + bp_ref[...]


@jax.jit
def kernel(x, Wg, bg, Wqkv, Wproj, bproj):
    B = x.shape[0]
    TB = 256

    qkv = pl.pallas_call(
        _qkv_body,
        grid=(NUM_EXPERTS, B // TB),
        in_specs=[
            pl.BlockSpec((TB, DIM), lambda e, t: (t, 0)),
            pl.BlockSpec((DIM, NUM_EXPERTS), lambda e, t: (0, 0)),
            pl.BlockSpec((1, NUM_EXPERTS), lambda e, t: (0, 0)),
            pl.BlockSpec((1, DIM, 3 * DIM), lambda e, t: (e, 0, 0)),
        ],
        out_specs=pl.BlockSpec((B, 3 * DIM), lambda e, t: (0, 0)),
        out_shape=jax.ShapeDtypeStruct((B, 3 * DIM), jnp.float32),
        scratch_shapes=[pltpu.VMEM((B, NUM_EXPERTS), jnp.float32)],
        compiler_params=pltpu.CompilerParams(
            dimension_semantics=("arbitrary", "arbitrary"),
        ),
    )(x, Wg, bg.reshape(1, NUM_EXPERTS), Wqkv)

    Wp2 = Wproj.reshape(DH, NUM_HEADS, DIM).transpose(1, 0, 2).reshape(DIM, DIM)

    out = pl.pallas_call(
        _attn_body,
        grid=(B // TB,),
        in_specs=[
            pl.BlockSpec((TB, 3 * DIM), lambda t: (t, 0)),
            pl.BlockSpec((DIM, DIM), lambda t: (0, 0)),
            pl.BlockSpec((1, DIM), lambda t: (0, 0)),
        ],
        out_specs=pl.BlockSpec((TB, DIM), lambda t: (t, 0)),
        out_shape=jax.ShapeDtypeStruct((B, DIM), jnp.float32),
        scratch_shapes=[pltpu.VMEM((TB, DIM), jnp.float32)],
        compiler_params=pltpu.CompilerParams(
            dimension_semantics=("parallel",),
        ),
    )(qkv, Wp2, bproj.reshape(1, DIM))

    return out


# speedup vs baseline: 1.3454x; 1.3454x over previous
import functools

import jax
import jax.numpy as jnp
from jax import lax
from jax.experimental import pallas as pl
from jax.experimental.pallas import tpu as pltpu
from jax.experimental.pallas import tpu_sc as plsc

DIM = 1024
NUM_EXPERTS = 8
NUM_HEADS = 16
TOP_K = 2
DH = DIM // NUM_HEADS
SCALE = DH ** (-0.5)

B = 2048
NPAIR = TOP_K * B
TBR = 256
NB = NPAIR // TBR + 8
NROWS = NB * TBR
NTILES = 32
PPT = NPAIR // NTILES
TPT = B // NTILES


def _route_body(x_ref, wg_ref, bg_ref, pos_ref, wp_ref, be_ref, cat_ref):
    logits = jnp.dot(x_ref[...], wg_ref[...],
                     preferred_element_type=jnp.float32) + bg_ref[...]
    T, E = logits.shape
    m = jnp.max(logits, axis=-1, keepdims=True)
    p = jnp.exp(logits - m)
    probs = p / jnp.sum(p, axis=-1, keepdims=True)

    idx = lax.broadcasted_iota(jnp.int32, (T, E), 1)
    big = jnp.int32(E)
    i1 = jnp.min(jnp.where(logits == m, idx, big), axis=-1, keepdims=True)
    mask1 = idx == i1
    logits2 = jnp.where(mask1, -jnp.inf, logits)
    max2 = jnp.max(logits2, axis=-1, keepdims=True)
    i2 = jnp.min(jnp.where(logits2 == max2, idx, big), axis=-1, keepdims=True)
    mask2 = idx == i2

    cat_ref[:T, :] = mask1.astype(jnp.float32)
    cat_ref[T:, :] = mask2.astype(jnp.float32)

    r = lax.broadcasted_iota(jnp.int32, (TBR, TBR), 0)
    c = lax.broadcasted_iota(jnp.int32, (TBR, TBR), 1)
    L = (c <= r).astype(jnp.float32)

    def step(i, carry):
        blk = cat_ref[pl.ds(i * TBR, TBR), :]
        cs = jnp.dot(L, blk, preferred_element_type=jnp.float32) + carry
        cat_ref[pl.ds(i * TBR, TBR), :] = cs
        return cs[TBR - 1:TBR, :]

    counts = lax.fori_loop(0, NPAIR // TBR, step,
                           jnp.zeros((1, E), jnp.float32))

    rounded = jnp.floor((counts + (TBR - 1)) / TBR) * TBR
    ue = lax.broadcasted_iota(jnp.int32, (E, E), 0)
    uc = lax.broadcasted_iota(jnp.int32, (E, E), 1)
    U = (ue < uc).astype(jnp.float32)
    ao = jnp.dot(rounded, U, preferred_element_type=jnp.float32)

    cum = cat_ref[...]
    ohm = jnp.concatenate([mask1, mask2], axis=0)
    posf = jnp.sum(jnp.where(ohm, ao + cum - 1.0, 0.0), axis=1, keepdims=True)
    pos_ref[...] = posf.astype(jnp.int32)

    probs2 = jnp.concatenate([probs, probs], axis=0)
    wp_ref[...] = jnp.sum(jnp.where(ohm, probs2, 0.0), axis=1, keepdims=True)

    aoc = jnp.transpose(ao)
    rc = jnp.transpose(rounded)
    eidx = lax.broadcasted_iota(jnp.int32, (E, NB), 0)
    bidx = lax.broadcasted_iota(jnp.int32, (E, NB), 1).astype(jnp.float32)
    rowpos = bidx * TBR
    mk = (rowpos >= aoc) & (rowpos < aoc + rc)
    be = jnp.sum(jnp.where(mk, eidx.astype(jnp.float32), 0.0), axis=0,
                 keepdims=True)
    be_ref[...] = be.astype(jnp.int32)


def _route(x, Wg, bg2):
    return pl.pallas_call(
        _route_body,
        in_specs=[
            pl.BlockSpec((B, DIM), lambda: (0, 0)),
            pl.BlockSpec((DIM, NUM_EXPERTS), lambda: (0, 0)),
            pl.BlockSpec((1, NUM_EXPERTS), lambda: (0, 0)),
        ],
        out_specs=[
            pl.BlockSpec((NPAIR, 1), lambda: (0, 0)),
            pl.BlockSpec((NPAIR, 1), lambda: (0, 0)),
            pl.BlockSpec((1, NB), lambda: (0, 0)),
        ],
        out_shape=[
            jax.ShapeDtypeStruct((NPAIR, 1), jnp.int32),
            jax.ShapeDtypeStruct((NPAIR, 1), jnp.float32),
            jax.ShapeDtypeStruct((1, NB), jnp.int32),
        ],
        scratch_shapes=[pltpu.VMEM((NPAIR, NUM_EXPERTS), jnp.float32)],
    )(x, Wg, bg2)


def _sc_scatter_body(x_hbm, pos3_hbm, xs_hbm, idx_v, rowA, rowB, semA, semB):
    cc = lax.axis_index("c")
    ss = lax.axis_index("s")
    wid = ss * 2 + cc
    base = wid * PPT
    tok0 = lax.rem(base, B)

    pltpu.sync_copy(pos3_hbm.at[wid], idx_v)

    copies = []
    for j in range(PPT // 16):
        buf = rowA if j % 2 == 0 else rowB
        sem = semA if j % 2 == 0 else semB
        if j >= 2:
            copies[j - 2].wait()
        pltpu.sync_copy(x_hbm.at[pl.ds(tok0 + j * 16, 16)], buf)
        copies.append(pltpu.async_copy(buf, xs_hbm.at[idx_v.at[j]], sem))
    copies[-2].wait()
    copies[-1].wait()


def _sc_scatter(x, pos3):
    mesh = plsc.VectorSubcoreMesh(core_axis_name="c", subcore_axis_name="s")
    f = pl.kernel(
        _sc_scatter_body,
        out_type=jax.ShapeDtypeStruct((NROWS, DIM), jnp.float32),
        mesh=mesh,
        scratch_types=[
            pltpu.VMEM((PPT // 16, 16), jnp.int32),
            pltpu.VMEM((16, DIM), jnp.float32),
            pltpu.VMEM((16, DIM), jnp.float32),
            pltpu.SemaphoreType.DMA,
            pltpu.SemaphoreType.DMA,
        ],
    )
    return f(x, pos3)


def _rmm_body(be_ref, xs_ref, wq_ref, out_ref):
    del be_ref
    xb = xs_ref[...].astype(jnp.bfloat16)
    out_ref[...] = jnp.dot(xb, wq_ref[0].astype(jnp.bfloat16),
                           preferred_element_type=jnp.float32)


def _rmm(be, xs, Wqkv):
    grid_spec = pltpu.PrefetchScalarGridSpec(
        num_scalar_prefetch=1,
        grid=(NB,),
        in_specs=[
            pl.BlockSpec((TBR, DIM), lambda b, bm: (b, 0)),
            pl.BlockSpec((1, DIM, 3 * DIM), lambda b, bm: (bm[b], 0, 0)),
        ],
        out_specs=pl.BlockSpec((TBR, 3 * DIM), lambda b, bm: (b, 0)),
    )
    return pl.pallas_call(
        _rmm_body,
        grid_spec=grid_spec,
        out_shape=jax.ShapeDtypeStruct((NROWS, 3 * DIM), jnp.float32),
        compiler_params=pltpu.CompilerParams(
            dimension_semantics=("arbitrary",),
        ),
    )(be, xs, Wqkv)


def _sc_combine_body(qkvs_hbm, pos1_hbm, pos2_hbm, outA_hbm, outB_hbm,
                     idx1_v, idx2_v, bufA, bufB, semA, semB):
    cc = lax.axis_index("c")
    ss = lax.axis_index("s")
    wid = ss * 2 + cc
    t0 = wid * TPT

    pltpu.sync_copy(pos1_hbm.at[wid], idx1_v)
    pltpu.sync_copy(pos2_hbm.at[wid], idx2_v)

    for j in range(TPT // 16):
        cpA = pltpu.async_copy(qkvs_hbm.at[idx1_v.at[j]], bufA, semA)
        cpB = pltpu.async_copy(qkvs_hbm.at[idx2_v.at[j]], bufB, semB)
        rows = pl.ds(t0 + j * 16, 16)
        cpA.wait()
        pltpu.sync_copy(bufA, outA_hbm.at[rows])
        cpB.wait()
        pltpu.sync_copy(bufB, outB_hbm.at[rows])


def _sc_combine(qkvs, pos1, pos2):
    mesh = plsc.VectorSubcoreMesh(core_axis_name="c", subcore_axis_name="s")
    f = pl.kernel(
        _sc_combine_body,
        out_type=[jax.ShapeDtypeStruct((B, 3 * DIM), jnp.float32),
                  jax.ShapeDtypeStruct((B, 3 * DIM), jnp.float32)],
        mesh=mesh,
        scratch_types=[
            pltpu.VMEM((TPT // 16, 16), jnp.int32),
            pltpu.VMEM((TPT // 16, 16), jnp.int32),
            pltpu.VMEM((16, 3 * DIM), jnp.float32),
            pltpu.VMEM((16, 3 * DIM), jnp.float32),
            pltpu.SemaphoreType.DMA,
            pltpu.SemaphoreType.DMA,
        ],
    )
    return f(qkvs, pos1, pos2)


def _attn_body(qkvA_ref, qkvB_ref, w1_ref, w2_ref, wp_ref, bp_ref,
               out_ref, att_ref):
    T = qkvA_ref.shape[0]
    ones = jnp.ones((1, 3 * DIM), jnp.float32)
    w1b = jnp.dot(w1_ref[...], ones, preferred_element_type=jnp.float32)
    w2b = jnp.dot(w2_ref[...], ones, preferred_element_type=jnp.float32)
    qkv = (w1b * qkvA_ref[...] + w2b * qkvB_ref[...]).astype(jnp.bfloat16)
    q = qkv[:, :DIM]
    kk = qkv[:, DIM:2 * DIM]
    v = qkv[:, 2 * DIM:]

    r = lax.broadcasted_iota(jnp.int32, (DIM, NUM_HEADS), 0)
    c = lax.broadcasted_iota(jnp.int32, (DIM, NUM_HEADS), 1)
    S = (r // DH == c).astype(jnp.bfloat16)
    r = lax.broadcasted_iota(jnp.int32, (NUM_HEADS, DIM), 0)
    c = lax.broadcasted_iota(jnp.int32, (NUM_HEADS, DIM), 1)
    St = (c // DH == r).astype(jnp.bfloat16)
    r = lax.broadcasted_iota(jnp.int32, (DIM, DH), 0)
    c = lax.broadcasted_iota(jnp.int32, (DIM, DH), 1)
    S2 = (r % DH == c).astype(jnp.bfloat16)
    r = lax.broadcasted_iota(jnp.int32, (DH, DIM), 0)
    c = lax.broadcasted_iota(jnp.int32, (DH, DIM), 1)
    S2t = (c % DH == r).astype(jnp.bfloat16)

    for i in range(NUM_HEADS):
        qi = q[:, i * DH:(i + 1) * DH]
        qrep = jnp.dot(qi, S2t,
                       preferred_element_type=jnp.float32).astype(jnp.bfloat16)
        logits = jnp.dot(qrep * kk, S,
                         preferred_element_type=jnp.float32) * SCALE
        logits = logits - jnp.max(logits, axis=-1, keepdims=True)
        w = jnp.exp(logits)
        w = (w / jnp.sum(w, axis=-1, keepdims=True)).astype(jnp.bfloat16)
        wrep = jnp.dot(w, St,
                       preferred_element_type=jnp.float32).astype(jnp.bfloat16)
        att_ref[:, i * DH:(i + 1) * DH] = jnp.dot(
            wrep * v, S2, preferred_element_type=jnp.float32)

    out_ref[...] = jnp.dot(att_ref[...].astype(jnp.bfloat16),
                           wp_ref[...].astype(jnp.bfloat16),
                           preferred_element_type=jnp.float32) + bp_ref[...]


def _attn(qkvA, qkvB, w1, w2, Wp2, bp2):
    TA = 512
    return pl.pallas_call(
        _attn_body,
        grid=(B // TA,),
        in_specs=[
            pl.BlockSpec((TA, 3 * DIM), lambda t: (t, 0)),
            pl.BlockSpec((TA, 3 * DIM), lambda t: (t, 0)),
            pl.BlockSpec((TA, 1), lambda t: (t, 0)),
            pl.BlockSpec((TA, 1), lambda t: (t, 0)),
            pl.BlockSpec((DIM, DIM), lambda t: (0, 0)),
            pl.BlockSpec((1, DIM), lambda t: (0, 0)),
        ],
        out_specs=pl.BlockSpec((TA, DIM), lambda t: (t, 0)),
        out_shape=jax.ShapeDtypeStruct((B, DIM), jnp.float32),
        scratch_shapes=[pltpu.VMEM((TA, DIM), jnp.float32)],
        compiler_params=pltpu.CompilerParams(
            dimension_semantics=("parallel",),
        ),
    )(qkvA, qkvB, w1, w2, Wp2, bp2)


@jax.jit
def kernel(x, Wg, bg, Wqkv, Wproj, bproj):
    posc, wc, be2 = _route(x, Wg, bg.reshape(1, NUM_EXPERTS))
    pos_flat = posc.reshape(NPAIR)
    pos3 = pos_flat.reshape(NTILES, PPT // 16, 16)
    pos1 = pos_flat[:B].reshape(NTILES, TPT // 16, 16)
    pos2 = pos_flat[B:].reshape(NTILES, TPT // 16, 16)
    be = be2.reshape(NB)

    xs = _sc_scatter(x, pos3)
    qkvs = _rmm(be, xs, Wqkv)
    qkvA, qkvB = _sc_combine(qkvs, pos1, pos2)

    Wp2 = Wproj.reshape(DH, NUM_HEADS, DIM).transpose(1, 0, 2).reshape(DIM, DIM)
    return _attn(qkvA, qkvB, wc[:B], wc[B:], Wp2, bproj.reshape(1, DIM))
